# baseline (speedup 1.0000x reference)
import functools
import jax
import jax.numpy as jnp
from jax import lax
from jax.experimental import pallas as pl
from jax.experimental.pallas import tpu as pltpu
from jax.experimental.pallas import tpu_sc as plsc

_N = 10000
_K = 32
_D = 128
_NS = 16
_BN = 1000
_G = 4
_MAXCNT = 320

_f32 = jnp.float32
_i32 = jnp.int32


def _uv_body(sw_ref, nw_ref, att_ref, out_ref):
    att = att_ref[...]
    u = jax.lax.dot_general(att, sw_ref[...], (((1,), (1,)), ((), ())),
                            preferred_element_type=_f32)
    v = jax.lax.dot_general(att, nw_ref[...], (((1,), (1,)), ((), ())),
                            preferred_element_type=_f32)
    out_ref[...] = jnp.concatenate([u, v], axis=0)


def _bcast(vec, j):
    return lax.gather(
        vec, jnp.full((16, 1), j, _i32),
        dimension_numbers=lax.GatherDimensionNumbers(
            offset_dims=(), collapsed_slice_dims=(0,),
            start_index_map=(0,)),
        slice_sizes=(1,),
        mode=lax.GatherScatterMode.PROMISE_IN_BOUNDS)


def _node_math(nb_k, self_row_chunks, u, v, iota, p_buf, zero16f, zero16i):
    for k in range(_K):
        acc = nb_k(k, 0) * v[0]
        for c in range(1, 8):
            acc = acc + nb_k(k, c) * v[c]
        p_buf[k, :] = acc

    sp = self_row_chunks[0] * u[0]
    for c in range(1, 8):
        sp = sp + self_row_chunks[c] * u[c]
    sl = jnp.maximum(jnp.sum(sp), 0.0)

    def halfsum(row0):
        cols = [plsc.load_gather(p_buf, [iota + row0,
                                         jnp.full((16,), c, _i32)])
                for c in range(16)]
        t = cols
        while len(t) > 1:
            t = [t[i] + t[i + 1] for i in range(0, len(t), 2)]
        return t[0]
    lo = jnp.maximum(halfsum(0), 0.0)
    hi = jnp.maximum(halfsum(16), 0.0)

    m = jnp.maximum(jnp.max(jnp.maximum(lo, hi)), sl)
    en_lo = jnp.exp(lo - m)
    en_hi = jnp.exp(hi - m)
    esv = jnp.exp(jnp.broadcast_to(sl - m, (16,)))
    z = jnp.broadcast_to(
        jnp.sum(en_lo + en_hi + jnp.where(iota == 0, esv, zero16f)),
        (16,))
    s_lo = en_lo / z
    s_hi = en_hi / z

    key_lo = jnp.bitwise_or(
        jnp.bitwise_and(lax.bitcast_convert_type(s_lo, _i32), ~31),
        31 - iota)
    key_hi = jnp.bitwise_or(
        jnp.bitwise_and(lax.bitcast_convert_type(s_hi, _i32), ~31),
        15 - iota)
    rank_lo = zero16i
    rank_hi = zero16i
    for keysrc in (key_lo, key_hi):
        for j in range(16):
            kj = _bcast(keysrc, j)
            rank_lo = rank_lo + (kj > key_lo).astype(_i32)
            rank_hi = rank_hi + (kj > key_hi).astype(_i32)
    w_lo = jnp.where(rank_lo < _NS, s_lo, zero16f)
    w_hi = jnp.where(rank_hi < _NS, s_hi, zero16f)

    comb = tuple(zero16f for _ in range(8))
    for koff, wsrc in ((0, w_lo), (16, w_hi)):
        for k in range(16):
            wk = _bcast(wsrc, k)
            comb = tuple(comb[c] + wk * nb_k(k + koff, c)
                         for c in range(8))
    return comb


def _sc_body(self_hbm, neigh_hbm, uv_hbm, out_hbm,
             nb_buf, uv_v, self_buf, p_buf, ob_buf, sem0, sem1):
    wid = lax.axis_index("s") * 2 + lax.axis_index("c")
    cnt = jnp.where(wid < 2, 320, 312)
    base = jnp.where(wid < 2, wid * 320, 640 + (wid - 2) * 312)
    sbase = jnp.minimum(base, _N - _MAXCNT)
    soff = base - sbase

    pltpu.sync_copy(uv_hbm, uv_v)
    pltpu.sync_copy(self_hbm.at[pl.ds(sbase, _MAXCNT)], self_buf)
    u = [uv_v[0, pl.ds(16 * c, 16)] for c in range(8)]
    v = [uv_v[1, pl.ds(16 * c, 16)] for c in range(8)]
    iota = lax.broadcasted_iota(_i32, (16,), 0)
    zero16f = jnp.zeros((16,), _f32)
    zero16i = jnp.zeros((16,), _i32)

    num_groups = (cnt + _G - 1) // _G

    def chunk_start(g):
        return jnp.minimum(g * _G, cnt - _G)

    sems = (sem0, sem1)

    def issue(g, slot):
        pltpu.async_copy(neigh_hbm.at[pl.ds(base + chunk_start(g), _G)],
                         nb_buf.at[slot], sems[slot])

    def wait_chunk(slot):
        pltpu.make_async_copy(neigh_hbm.at[pl.ds(0, _G)],
                              nb_buf.at[slot], sems[slot]).wait()

    issue(0, 0)
    issue(1, 1)
    npairs = (num_groups + 1) // 2

    def pair_body(p, _):
        for b in range(2):
            g = 2 * p + b
            gs = chunk_start(g)
            wait_chunk(b)
            for q in range(_G):
                n_local = gs + q

                def nb_k(k, c):
                    return nb_buf[b, q, k, pl.ds(16 * c, 16)]
                srow = [self_buf[soff + n_local, pl.ds(16 * c, 16)]
                        for c in range(8)]
                comb = _node_math(nb_k, srow, u, v, iota, p_buf,
                                  zero16f, zero16i)
                for c in range(8):
                    ob_buf[q, pl.ds(16 * c, 16)] = comb[c]
            pltpu.sync_copy(ob_buf, out_hbm.at[pl.ds(base + gs, _G)])
            issue(g + 2, b)
        return 0

    lax.fori_loop(0, npairs, pair_body, 0)
    wait_chunk(0)
    wait_chunk(1)


def _sc_combined(self_vecs, neigh_vecs, uv):
    mesh = plsc.VectorSubcoreMesh(core_axis_name="c", subcore_axis_name="s")
    kfn = functools.partial(
        pl.kernel,
        out_type=jax.ShapeDtypeStruct((_N, _D), _f32),
        mesh=mesh,
        scratch_types=[
            pltpu.VMEM((2, _G, _K, _D), _f32),
            pltpu.VMEM((2, _D), _f32),
            pltpu.VMEM((_MAXCNT, _D), _f32),
            pltpu.VMEM((_K, 16), _f32),
            pltpu.VMEM((_G, _D), _f32),
            pltpu.SemaphoreType.DMA,
            pltpu.SemaphoreType.DMA,
        ],
        compiler_params=pltpu.CompilerParams(needs_layout_passes=False),
    )(_sc_body)
    return kfn(self_vecs, neigh_vecs, uv)


def _final_body(self_ref, comb_ref, sw_ref, nw_ref, out_ref):
    st = jax.lax.dot_general(self_ref[...], sw_ref[...],
                             (((1,), (0,)), ((), ())),
                             preferred_element_type=_f32)
    cn = jax.lax.dot_general(comb_ref[...], nw_ref[...],
                             (((1,), (0,)), ((), ())),
                             preferred_element_type=_f32)
    out_ref[...] = jax.nn.relu(st + cn)


def kernel(self_vecs, neigh_vecs, self_weights, neigh_weights,
           attention_weights):
    att = attention_weights.reshape(1, _D)
    uv = pl.pallas_call(
        _uv_body,
        out_shape=jax.ShapeDtypeStruct((2, _D), _f32),
    )(self_weights, neigh_weights, att)
    comb = _sc_combined(self_vecs, neigh_vecs, uv)
    return pl.pallas_call(
        _final_body,
        grid=(_N // _BN,),
        in_specs=[
            pl.BlockSpec((_BN, _D), lambda i: (i, 0)),
            pl.BlockSpec((_BN, _D), lambda i: (i, 0)),
            pl.BlockSpec((_D, _D), lambda i: (0, 0)),
            pl.BlockSpec((_D, _D), lambda i: (0, 0)),
        ],
        out_specs=pl.BlockSpec((_BN, _D), lambda i: (i, 0)),
        out_shape=jax.ShapeDtypeStruct((_N, _D), jnp.float32),
        compiler_params=pltpu.CompilerParams(
            dimension_semantics=("arbitrary",),
        ),
    )(self_vecs, comb, self_weights, neigh_weights)


# speedup vs baseline: 2.4068x; 2.4068x over previous
import functools
import jax
import jax.numpy as jnp
from jax import lax
from jax.experimental import pallas as pl
from jax.experimental.pallas import tpu as pltpu
from jax.experimental.pallas import tpu_sc as plsc

_N = 10000
_K = 32
_D = 128
_NS = 16
_BN = 1000
_G = 4
_MAXCNT = 320

_f32 = jnp.float32
_i32 = jnp.int32


def _uv_body(sw_ref, nw_ref, att_ref, out_ref):
    att = att_ref[...]
    u = jax.lax.dot_general(att, sw_ref[...], (((1,), (1,)), ((), ())),
                            preferred_element_type=_f32)
    v = jax.lax.dot_general(att, nw_ref[...], (((1,), (1,)), ((), ())),
                            preferred_element_type=_f32)
    out_ref[...] = jnp.concatenate([u, v], axis=0)


def _bcast(vec, j):
    return lax.gather(
        vec, jnp.full((16, 1), j, _i32),
        dimension_numbers=lax.GatherDimensionNumbers(
            offset_dims=(), collapsed_slice_dims=(0,),
            start_index_map=(0,)),
        slice_sizes=(1,),
        mode=lax.GatherScatterMode.PROMISE_IN_BOUNDS)


def _node_math(nb_k, self_row_chunks, u, v, iota, p_buf, zero16f, zero16i):
    for k in range(_K):
        acc = nb_k(k, 0) * v[0]
        for c in range(1, 8):
            acc = acc + nb_k(k, c) * v[c]
        p_buf[k, :] = acc

    sp = self_row_chunks[0] * u[0]
    for c in range(1, 8):
        sp = sp + self_row_chunks[c] * u[c]
    sl = jnp.maximum(jnp.sum(sp), 0.0)

    def halfsum(row0):
        cols = [plsc.load_gather(p_buf, [iota + row0,
                                         jnp.full((16,), c, _i32)])
                for c in range(16)]
        t = cols
        while len(t) > 1:
            t = [t[i] + t[i + 1] for i in range(0, len(t), 2)]
        return t[0]
    lo = jnp.maximum(halfsum(0), 0.0)
    hi = jnp.maximum(halfsum(16), 0.0)

    m = jnp.maximum(jnp.max(jnp.maximum(lo, hi)), sl)
    en_lo = jnp.exp(lo - m)
    en_hi = jnp.exp(hi - m)
    esv = jnp.exp(jnp.broadcast_to(sl - m, (16,)))
    z = jnp.broadcast_to(
        jnp.sum(en_lo + en_hi + jnp.where(iota == 0, esv, zero16f)),
        (16,))
    s_lo = en_lo / z
    s_hi = en_hi / z

    key_lo = jnp.bitwise_or(
        jnp.bitwise_and(lax.bitcast_convert_type(s_lo, _i32), ~31),
        31 - iota)
    key_hi = jnp.bitwise_or(
        jnp.bitwise_and(lax.bitcast_convert_type(s_hi, _i32), ~31),
        15 - iota)
    rank_lo = zero16i
    rank_hi = zero16i
    for keysrc in (key_lo, key_hi):
        for j in range(16):
            kj = _bcast(keysrc, j)
            rank_lo = rank_lo + (kj > key_lo).astype(_i32)
            rank_hi = rank_hi + (kj > key_hi).astype(_i32)
    w_lo = jnp.where(rank_lo < _NS, s_lo, zero16f)
    w_hi = jnp.where(rank_hi < _NS, s_hi, zero16f)

    comb = tuple(zero16f for _ in range(8))
    for koff, wsrc in ((0, w_lo), (16, w_hi)):
        for k in range(16):
            wk = _bcast(wsrc, k)
            comb = tuple(comb[c] + wk * nb_k(k + koff, c)
                         for c in range(8))
    return comb


def _sc_body(self_hbm, neigh_hbm, uv_hbm, out_hbm,
             nb_buf, uv_v, self_buf, p_buf, ob_buf, sem0, sem1):
    wid = lax.axis_index("s") * 2 + lax.axis_index("c")
    cnt = jnp.where(wid < 2, 320, 312)
    base = jnp.where(wid < 2, wid * 320, 640 + (wid - 2) * 312)
    sbase = jnp.minimum(base, _N - _MAXCNT)
    soff = base - sbase

    pltpu.sync_copy(uv_hbm, uv_v)
    pltpu.sync_copy(self_hbm.at[pl.ds(sbase, _MAXCNT)], self_buf)
    u = [uv_v[0, pl.ds(16 * c, 16)] for c in range(8)]
    v = [uv_v[1, pl.ds(16 * c, 16)] for c in range(8)]
    iota = lax.broadcasted_iota(_i32, (16,), 0)
    zero16f = jnp.zeros((16,), _f32)
    zero16i = jnp.zeros((16,), _i32)

    num_groups = (cnt + _G - 1) // _G

    def chunk_start(g):
        return jnp.minimum(g * _G, cnt - _G)

    sems = (sem0, sem1)

    def issue(g, slot):
        pltpu.async_copy(neigh_hbm.at[pl.ds(base + chunk_start(g), _G)],
                         nb_buf.at[slot], sems[slot])

    def wait_chunk(slot):
        pltpu.make_async_copy(neigh_hbm.at[pl.ds(0, _G)],
                              nb_buf.at[slot], sems[slot]).wait()

    issue(0, 0)
    issue(1, 1)
    npairs = (num_groups + 1) // 2

    def pair_body(p, _):
        for b in range(2):
            g = 2 * p + b
            gs = chunk_start(g)
            wait_chunk(b)
            for q in range(_G):
                n_local = gs + q

                def nb_k(k, c):
                    return nb_buf[b, q, k, pl.ds(16 * c, 16)]
                srow = [self_buf[soff + n_local, pl.ds(16 * c, 16)]
                        for c in range(8)]
                comb = _node_math(nb_k, srow, u, v, iota, p_buf,
                                  zero16f, zero16i)
                for c in range(8):
                    ob_buf[q, pl.ds(16 * c, 16)] = comb[c]
            pltpu.sync_copy(ob_buf, out_hbm.at[pl.ds(base + gs, _G)])
            issue(g + 2, b)
        return 0

    lax.fori_loop(0, npairs, pair_body, 0)
    wait_chunk(0)
    wait_chunk(1)


def _sc_combined(self_vecs, neigh_vecs, uv):
    mesh = plsc.VectorSubcoreMesh(core_axis_name="c", subcore_axis_name="s")
    kfn = functools.partial(
        pl.kernel,
        out_type=jax.ShapeDtypeStruct((_N, _D), _f32),
        mesh=mesh,
        scratch_types=[
            pltpu.VMEM((2, _G, _K, _D), _f32),
            pltpu.VMEM((2, _D), _f32),
            pltpu.VMEM((_MAXCNT, _D), _f32),
            pltpu.VMEM((_K, 16), _f32),
            pltpu.VMEM((_G, _D), _f32),
            pltpu.SemaphoreType.DMA,
            pltpu.SemaphoreType.DMA,
        ],
        compiler_params=pltpu.CompilerParams(needs_layout_passes=False),
    )(_sc_body)
    return kfn(self_vecs, neigh_vecs, uv)


def _final_body(self_ref, comb_ref, sw_ref, nw_ref, out_ref):
    st = jax.lax.dot_general(self_ref[...], sw_ref[...],
                             (((1,), (0,)), ((), ())),
                             preferred_element_type=_f32)
    cn = jax.lax.dot_general(comb_ref[...], nw_ref[...],
                             (((1,), (0,)), ((), ())),
                             preferred_element_type=_f32)
    out_ref[...] = jax.nn.relu(st + cn)


def kernel(self_vecs, neigh_vecs, self_weights, neigh_weights,
           attention_weights):
    att = attention_weights.reshape(1, _D)
    uv = pl.pallas_call(
        _uv_body,
        out_shape=jax.ShapeDtypeStruct((2, _D), _f32),
    )(self_weights, neigh_weights, att)
    comb = _sc_combined(self_vecs, neigh_vecs, uv)
    return pl.pallas_call(
        _final_body,
        grid=(_N // _BN,),
        in_specs=[
            pl.BlockSpec((_BN, _D), lambda i: (i, 0)),
            pl.BlockSpec((_BN, _D), lambda i: (i, 0)),
            pl.BlockSpec((_D, _D), lambda i: (0, 0)),
            pl.BlockSpec((_D, _D), lambda i: (0, 0)),
        ],
        out_specs=pl.BlockSpec((_BN, _D), lambda i: (i, 0)),
        out_shape=jax.ShapeDtypeStruct((_N, _D), jnp.float32),
        compiler_params=pltpu.CompilerParams(
            dimension_semantics=("arbitrary",),
        ),
    )(self_vecs, comb, self_weights, neigh_weights)


import kernel_hy as _khy
kernel = _khy.kernel_hy


# speedup vs baseline: 2.4471x; 1.0167x over previous
import functools
import jax
import jax.numpy as jnp
from jax import lax
from jax.experimental import pallas as pl
from jax.experimental.pallas import tpu as pltpu
from jax.experimental.pallas import tpu_sc as plsc

_N = 10000
_M = 2560
_NT = _N - _M
_PT = _M // 32
_K = 32
_D = 128
_NS = 16
_f32 = jnp.float32
_i32 = jnp.int32
_G = 8
_BN_TC = 496
_BN_F = 512


def _fused_body_bn(_BN, self_ref, neigh_ref, sw_ref, nw_ref, att_ref, out_ref):
    sv = self_ref[...]
    nb = neigh_ref[...]
    sw = sw_ref[...]
    nw = nw_ref[...]
    att = att_ref[...]

    f32 = jnp.float32
    u = jax.lax.dot_general(sw, att, (((1,), (1,)), ((), ())),
                            preferred_element_type=f32)
    v = jax.lax.dot_general(nw, att, (((1,), (1,)), ((), ())),
                            preferred_element_type=f32)

    self_logit = jax.nn.relu(
        jax.lax.dot_general(sv, u, (((1,), (0,)), ((), ())),
                            preferred_element_type=f32))
    nl = jax.lax.dot_general(nb.reshape(_BN * _K, _D), v,
                             (((1,), (0,)), ((), ())),
                             preferred_element_type=f32)
    neigh_logits = jax.nn.relu(nl.reshape(_BN, _K))

    lt = neigh_logits.T
    st = self_logit.T

    m = jnp.maximum(jnp.max(lt, axis=0, keepdims=True), st)
    en = jnp.exp(lt - m)
    es = jnp.exp(st - m)
    z = es + jnp.sum(en, axis=0, keepdims=True)
    s = en / z

    iota_k = jax.lax.broadcasted_iota(jnp.int32, (_K, _BN), 0)
    key = jnp.bitwise_or(
        jnp.bitwise_and(jax.lax.bitcast_convert_type(s, jnp.int32), ~31),
        (_K - 1) - iota_k)
    rank = jnp.zeros((_K, _BN), dtype=jnp.int32)
    for j in range(_K):
        row = key[j:j + 1, :]
        rank = rank + (row > key).astype(jnp.int32)
    w = jnp.where(rank < _NS, s, 0.0)

    wt = w.T
    combined = jnp.sum(wt[:, :, None] * nb, axis=1)
    st_out = jax.lax.dot_general(sv, sw, (((1,), (0,)), ((), ())),
                                 preferred_element_type=f32)
    cn = jax.lax.dot_general(combined, nw, (((1,), (0,)), ((), ())),
                             preferred_element_type=f32)
    out_ref[...] = jax.nn.relu(st_out + cn)


def _uv_body(sw_ref, nw_ref, att_ref, out_ref):
    att = att_ref[...]
    u = jax.lax.dot_general(att, sw_ref[...], (((1,), (1,)), ((), ())),
                            preferred_element_type=_f32)
    v = jax.lax.dot_general(att, nw_ref[...], (((1,), (1,)), ((), ())),
                            preferred_element_type=_f32)
    out_ref[...] = jnp.concatenate([u, v], axis=0)


def _bcast(vec, j):
    return lax.gather(
        vec, jnp.full((16, 1), j, _i32),
        dimension_numbers=lax.GatherDimensionNumbers(
            offset_dims=(), collapsed_slice_dims=(0,),
            start_index_map=(0,)),
        slice_sizes=(1,),
        mode=lax.GatherScatterMode.PROMISE_IN_BOUNDS)


def _node_math(nb_k, self_row_chunks, u, v, iota, p_buf, zero16f, zero16i):
    def k_body(k, carry):
        acc = nb_k(k, 0) * v[0]
        for c in range(1, 8):
            acc = acc + nb_k(k, c) * v[c]
        p_buf[k, :] = acc
        return carry
    lax.fori_loop(0, _K, k_body, 0, unroll=8)

    sp = self_row_chunks[0] * u[0]
    for c in range(1, 8):
        sp = sp + self_row_chunks[c] * u[c]
    sl = jnp.maximum(jnp.sum(sp), 0.0)

    def halfsum(row0):
        cols = [plsc.load_gather(p_buf, [iota + row0,
                                         jnp.full((16,), c, _i32)])
                for c in range(16)]
        t = cols
        while len(t) > 1:
            t = [t[i] + t[i + 1] for i in range(0, len(t), 2)]
        return t[0]
    lo = jnp.maximum(halfsum(0), 0.0)
    hi = jnp.maximum(halfsum(16), 0.0)

    m = jnp.maximum(jnp.max(jnp.maximum(lo, hi)), sl)
    en_lo = jnp.exp(lo - m)
    en_hi = jnp.exp(hi - m)
    esv = jnp.exp(jnp.broadcast_to(sl - m, (16,)))
    z = jnp.broadcast_to(
        jnp.sum(en_lo + en_hi + jnp.where(iota == 0, esv, zero16f)),
        (16,))
    s_lo = en_lo / z
    s_hi = en_hi / z

    key_lo = jnp.bitwise_or(
        jnp.bitwise_and(lax.bitcast_convert_type(s_lo, _i32), ~31),
        31 - iota)
    key_hi = jnp.bitwise_or(
        jnp.bitwise_and(lax.bitcast_convert_type(s_hi, _i32), ~31),
        15 - iota)

    def j_body_src(keysrc):
        def j_body(j, carry):
            rl, rh = carry
            kj = _bcast(keysrc, j)
            rl = rl + (kj > key_lo).astype(_i32)
            rh = rh + (kj > key_hi).astype(_i32)
            return rl, rh
        return j_body
    rank = lax.fori_loop(0, 16, j_body_src(key_lo), (zero16i, zero16i),
                         unroll=8)
    rank_lo, rank_hi = lax.fori_loop(0, 16, j_body_src(key_hi), rank,
                                     unroll=8)
    w_lo = jnp.where(rank_lo < _NS, s_lo, zero16f)
    w_hi = jnp.where(rank_hi < _NS, s_hi, zero16f)

    def wk_body_src(wsrc, koff):
        def wk_body(k, comb):
            wk = _bcast(wsrc, k)
            return tuple(comb[c] + wk * nb_k(k + koff, c)
                         for c in range(8))
        return wk_body
    comb = lax.fori_loop(0, 16, wk_body_src(w_lo, 0),
                         tuple(zero16f for _ in range(8)), unroll=8)
    comb = lax.fori_loop(0, 16, wk_body_src(w_hi, 16), comb, unroll=8)
    return comb


def _sc_body(self_hbm, neigh_hbm, uv_hbm, out_hbm,
             nb_buf, uv_v, self_buf, p_buf, ob_buf, sem0, sem1):
    wid = lax.axis_index("s") * 2 + lax.axis_index("c")
    base = _NT + wid * _PT

    pltpu.sync_copy(uv_hbm, uv_v)
    pltpu.sync_copy(self_hbm.at[pl.ds(base, _PT)], self_buf)
    u = [uv_v[0, pl.ds(16 * c, 16)] for c in range(8)]
    v = [uv_v[1, pl.ds(16 * c, 16)] for c in range(8)]
    iota = lax.broadcasted_iota(_i32, (16,), 0)
    zero16f = jnp.zeros((16,), _f32)
    zero16i = jnp.zeros((16,), _i32)

    sems = (sem0, sem1)
    ngroups = _PT // _G

    def issue(g, slot):
        g = jnp.minimum(g, ngroups - 1)
        pltpu.async_copy(neigh_hbm.at[pl.ds(base + g * _G, _G)],
                         nb_buf.at[slot], sems[slot])

    def wait_chunk(slot):
        pltpu.make_async_copy(neigh_hbm.at[pl.ds(0, _G)],
                              nb_buf.at[slot], sems[slot]).wait()

    issue(0, 0)
    issue(1, 1)

    def pair_body(p, _):
        for b in range(2):
            g = 2 * p + b
            gs = g * _G
            wait_chunk(b)
            for q in range(_G):
                n_local = gs + q

                def nb_k(k, c):
                    return nb_buf[b, q, k, pl.ds(16 * c, 16)]
                srow = [self_buf[n_local, pl.ds(16 * c, 16)]
                        for c in range(8)]
                comb = _node_math(nb_k, srow, u, v, iota, p_buf,
                                  zero16f, zero16i)
                for c in range(8):
                    ob_buf[q, pl.ds(16 * c, 16)] = comb[c]
            pltpu.sync_copy(ob_buf, out_hbm.at[pl.ds(wid * _PT + gs, _G)])
            issue(g + 2, b)
        return 0

    lax.fori_loop(0, ngroups // 2, pair_body, 0)
    wait_chunk(0)
    wait_chunk(1)


def _sc_combined(self_vecs, neigh_vecs, uv):
    mesh = plsc.VectorSubcoreMesh(core_axis_name="c", subcore_axis_name="s")
    kfn = functools.partial(
        pl.kernel,
        out_type=jax.ShapeDtypeStruct((_M, _D), _f32),
        mesh=mesh,
        scratch_types=[
            pltpu.VMEM((2, _G, _K, _D), _f32),
            pltpu.VMEM((2, _D), _f32),
            pltpu.VMEM((_PT, _D), _f32),
            pltpu.VMEM((_K, 16), _f32),
            pltpu.VMEM((_G, _D), _f32),
            pltpu.SemaphoreType.DMA,
            pltpu.SemaphoreType.DMA,
        ],
        compiler_params=pltpu.CompilerParams(needs_layout_passes=False),
    )(_sc_body)
    return kfn(self_vecs, neigh_vecs, uv)


def _final_body(self_ref, comb_ref, sw_ref, nw_ref, out_ref):
    st = jax.lax.dot_general(self_ref[...], sw_ref[...],
                             (((1,), (0,)), ((), ())),
                             preferred_element_type=_f32)
    cn = jax.lax.dot_general(comb_ref[...], nw_ref[...],
                             (((1,), (0,)), ((), ())),
                             preferred_element_type=_f32)
    out_ref[...] = jax.nn.relu(st + cn)


def _tc_fused(self_vecs, neigh_vecs, self_weights, neigh_weights, att):
    body = functools.partial(_fused_body_bn, _BN_TC)
    return pl.pallas_call(
        body,
        grid=(_NT // _BN_TC,),
        in_specs=[
            pl.BlockSpec((_BN_TC, _D), lambda i: (i, 0)),
            pl.BlockSpec((_BN_TC, _K, _D), lambda i: (i, 0, 0)),
            pl.BlockSpec((_D, _D), lambda i: (0, 0)),
            pl.BlockSpec((_D, _D), lambda i: (0, 0)),
            pl.BlockSpec((1, _D), lambda i: (0, 0)),
        ],
        out_specs=pl.BlockSpec((_BN_TC, _D), lambda i: (i, 0)),
        out_shape=jax.ShapeDtypeStruct((_NT, _D), jnp.float32),
        compiler_params=pltpu.CompilerParams(
            dimension_semantics=("arbitrary",),
        ),
    )(self_vecs[:_NT], neigh_vecs[:_NT], self_weights, neigh_weights, att)


def kernel(self_vecs, neigh_vecs, self_weights, neigh_weights,
              attention_weights):
    att = attention_weights.reshape(1, _D)
    uv = pl.pallas_call(
        _uv_body,
        out_shape=jax.ShapeDtypeStruct((2, _D), _f32),
    )(self_weights, neigh_weights, att)
    comb_sc = _sc_combined(self_vecs, neigh_vecs, uv)
    out_tc = _tc_fused(self_vecs, neigh_vecs, self_weights, neigh_weights,
                       att)
    out_sc = pl.pallas_call(
        _final_body,
        grid=(_M // _BN_F,),
        in_specs=[
            pl.BlockSpec((_BN_F, _D), lambda i: (i, 0)),
            pl.BlockSpec((_BN_F, _D), lambda i: (i, 0)),
            pl.BlockSpec((_D, _D), lambda i: (0, 0)),
            pl.BlockSpec((_D, _D), lambda i: (0, 0)),
        ],
        out_specs=pl.BlockSpec((_BN_F, _D), lambda i: (i, 0)),
        out_shape=jax.ShapeDtypeStruct((_M, _D), jnp.float32),
        compiler_params=pltpu.CompilerParams(
            dimension_semantics=("arbitrary",),
        ),
    )(self_vecs[_NT:], comb_sc, self_weights, neigh_weights)
    return jnp.concatenate([out_tc, out_sc], axis=0)
